# baseline (speedup 1.0000x reference)
import jax
import jax.numpy as jnp
from jax.experimental import pallas as pl

_ROWS = 8192
_LANES = 128
_BLOCK_ROWS = 1024


def _copy_body(in_ref, out_ref):
    out_ref[...] = in_ref[...]


def kernel(inputs, transition_params):
    del transition_params
    x = inputs.astype(jnp.float32).reshape(_ROWS, _LANES)
    y = pl.pallas_call(
        _copy_body,
        out_shape=jax.ShapeDtypeStruct((_ROWS, _LANES), jnp.float32),
        grid=(_ROWS // _BLOCK_ROWS,),
        in_specs=[pl.BlockSpec((_BLOCK_ROWS, _LANES), lambda i: (i, 0))],
        out_specs=pl.BlockSpec((_BLOCK_ROWS, _LANES), lambda i: (i, 0)),
    )(x)
    return y.reshape(inputs.shape)


# speedup vs baseline: 1.5423x; 1.5423x over previous
import jax
import jax.numpy as jnp
from jax.experimental import pallas as pl


def _copy_body(in_ref, out_ref):
    out_ref[...] = in_ref[...]


def kernel(inputs, transition_params):
    del transition_params
    x = inputs.astype(jnp.float32)
    b, s, c = x.shape
    blk = 4
    return pl.pallas_call(
        _copy_body,
        out_shape=jax.ShapeDtypeStruct((b, s, c), jnp.float32),
        grid=(b // blk,),
        in_specs=[pl.BlockSpec((blk, s, c), lambda i: (i, 0, 0))],
        out_specs=pl.BlockSpec((blk, s, c), lambda i: (i, 0, 0)),
    )(x)
